# baseline (speedup 1.0000x reference)
import functools

import jax
import jax.numpy as jnp
from jax import lax
from jax.experimental import pallas as pl
from jax.experimental.pallas import tpu as pltpu
from jax.experimental.pallas import tpu_sc as plsc

N = 10000
E = 320000
D = 128
EPS = 1e-7
MIN_NORM = 1e-15

NC = 2
NS = 16
K = 96
NCH = 108
EPT = NCH * K
EPAD = NC * NS * EPT
NSLOT = 3
NLSL = 6
RPT = 624
ZR = 16
NZC = RPT // ZR


def _spmm_body(mat_hbm, src_hbm, dst_hbm, val_hbm, out_hbm,
               acc, sidx, didx, vals, rows, zbuf,
               lsem0, lsem1, lsem2, lsem3, lsem4, lsem5,
               gsem0, gsem1, gsem2, ssem0, ssem1, ssem2, zsem):
    cid = lax.axis_index("c")
    sid = lax.axis_index("s")
    lsem = (lsem0, lsem1, lsem2, lsem3, lsem4, lsem5)
    gsem = (gsem0, gsem1, gsem2)
    ssem = (ssem0, ssem1, ssem2)

    tile = cid * NS + sid
    ebase = tile * EPT
    row0 = sid * RPT
    last = sid == NS - 1

    def issue_load(c, u):
        off = ebase + c * K
        pltpu.async_copy(src_hbm.at[pl.ds(off, K)], sidx.at[u], lsem[u])
        pltpu.async_copy(dst_hbm.at[pl.ds(off, K)], didx.at[u], lsem[u])
        pltpu.async_copy(val_hbm.at[pl.ds(off, K)], vals.at[u], lsem[u])

    def wait_load(c, u):
        off = ebase + c * K
        pltpu.make_async_copy(src_hbm.at[pl.ds(off, K)], sidx.at[u], lsem[u]).wait()
        pltpu.make_async_copy(dst_hbm.at[pl.ds(off, K)], didx.at[u], lsem[u]).wait()
        pltpu.make_async_copy(val_hbm.at[pl.ds(off, K)], vals.at[u], lsem[u]).wait()

    def issue_gather(u, s):
        pltpu.async_copy(mat_hbm.at[sidx.at[u]], rows.at[s], gsem[s])

    def wait_gather(u, s):
        pltpu.make_async_copy(mat_hbm.at[sidx.at[u]], rows.at[s], gsem[s]).wait()

    def issue_scatter(u, s):
        pltpu.async_copy(rows.at[s], acc.at[didx.at[u]], ssem[s], add=True)

    def wait_scatter(s):
        pltpu.make_async_copy(rows.at[s], acc.at[didx.at[0]], ssem[s]).wait()

    def scale(u, s):
        def group(g, carry):
            vv = vals[u, pl.ds(g * 16, 16)]
            for l in range(16):
                r = g * 16 + l
                sv = jnp.broadcast_to(vv[l], (16,))
                for q in range(D // 16):
                    sl = pl.ds(q * 16, 16)
                    rows[s, r, sl] = rows[s, r, sl] * sv
            return carry

        lax.fori_loop(0, K // 16, group, 0)

    z16 = jnp.zeros((16,), jnp.float32)
    for r in range(ZR):
        for q in range(D // 16):
            zbuf[r, pl.ds(q * 16, 16)] = z16
    zds = [pltpu.async_copy(zbuf, acc.at[pl.ds(row0 + k * ZR, ZR), :], zsem)
           for k in range(NZC)]

    for c in range(4):
        issue_load(c, c)
    for c in range(2):
        wait_load(c, c)
        issue_gather(c, c)

    for d in zds:
        d.wait()

    @pl.when(last)
    def _():
        pltpu.sync_copy(zbuf, acc.at[pl.ds(row0 + NZC * ZR, ZR), :])

    plsc.subcore_barrier()

    def round_(k, carry):
        t0 = k * NLSL
        for uu in range(NLSL):
            t = t0 + uu

            @pl.when(t + 4 < NCH)
            def _():
                issue_load(t + 4, (uu + 4) % NLSL)

            @pl.when(t + 2 < NCH)
            def _():
                if uu == 0:
                    @pl.when(k > 0)
                    def _():
                        wait_scatter((uu + 2) % NSLOT)
                else:
                    wait_scatter((uu + 2) % NSLOT)
                wait_load(t + 2, (uu + 2) % NLSL)
                issue_gather((uu + 2) % NLSL, (uu + 2) % NSLOT)

            wait_gather(uu, uu % NSLOT)
            scale(uu, uu % NSLOT)
            issue_scatter(uu, uu % NSLOT)
        return carry

    lax.fori_loop(0, NCH // NLSL, round_, 0)

    for s in range(NSLOT):
        wait_scatter(s)

    plsc.subcore_barrier()

    wds = [pltpu.async_copy(acc.at[pl.ds(row0 + k * ZR, ZR), :],
                            out_hbm.at[cid, pl.ds(row0 + k * ZR, ZR), :], zsem)
           for k in range(NZC)]

    @pl.when(last)
    def _():
        r0 = row0 + NZC * ZR
        pltpu.sync_copy(acc.at[pl.ds(r0, ZR), :], out_hbm.at[cid, pl.ds(r0, ZR), :])

    for d in wds:
        d.wait()


def _spmm_sc(mat, srcp, dstp, valp):
    mesh = plsc.VectorSubcoreMesh(
        core_axis_name="c", subcore_axis_name="s", num_cores=NC, num_subcores=NS)
    f = pl.kernel(
        _spmm_body,
        out_type=jax.ShapeDtypeStruct((NC, N, D), jnp.float32),
        mesh=mesh,
        scratch_types=[
            pltpu.VMEM_SHARED((N, D), jnp.float32),
            pltpu.VMEM((NLSL, K), jnp.int32),
            pltpu.VMEM((NLSL, K), jnp.int32),
            pltpu.VMEM((NLSL, K), jnp.float32),
            pltpu.VMEM((NSLOT, K, D), jnp.float32),
            pltpu.VMEM((ZR, D), jnp.float32),
        ] + [pltpu.SemaphoreType.DMA] * 13,
    )
    return f(mat, srcp, dstp, valp)


def _row_block_call(body, *args):
    rows = 1000
    grid = (N // rows,)
    spec = pl.BlockSpec((rows, D), lambda i: (i, 0))
    out_shape = jax.ShapeDtypeStruct((N, D), jnp.float32)
    return pl.pallas_call(
        body, out_shape=out_shape, grid=grid,
        in_specs=[spec] * len(args), out_specs=spec)(*args)


def _tangent_body(x_ref, o_ref):
    xb = x_ref[...]
    col = lax.broadcasted_iota(jnp.int32, xb.shape, 1)
    xm = jnp.where(col > 0, xb, 0.0)
    s = jnp.sum(xm * xm, axis=1, keepdims=True)
    t = jnp.sqrt(1.0 + s)
    theta = jnp.maximum(t, 1.0 + EPS)
    yn = jnp.maximum(jnp.sqrt(s), MIN_NORM)
    coef = jnp.log(theta + jnp.sqrt(theta * theta - 1.0)) / yn
    o_ref[...] = xm * coef


def _add_body(a_ref, b_ref, o_ref):
    o_ref[...] = a_ref[...] + b_ref[...]


def _final_body(h_ref, a_ref, b_ref, o_ref):
    u = h_ref[...] + a_ref[...] + b_ref[...]
    col = lax.broadcasted_iota(jnp.int32, u.shape, 1)
    um = jnp.where(col > 0, u, 0.0)
    s = jnp.sum(um * um, axis=1, keepdims=True)
    xn = jnp.maximum(jnp.sqrt(s), MIN_NORM)
    e = jnp.exp(xn)
    sinh = 0.5 * (e - 1.0 / e)
    sp = (sinh / xn) * um
    s2 = jnp.sum(sp * sp, axis=1, keepdims=True)
    t2 = jnp.sqrt(jnp.maximum(1.0 + s2, EPS))
    o_ref[...] = jnp.where(col > 0, sp, t2)


def kernel(x, edge_index, adj_values):
    pad = EPAD - E
    dstp = jnp.pad(edge_index[0], (0, pad))
    srcp = jnp.pad(edge_index[1], (0, pad))
    valp = jnp.pad(adj_values, (0, pad))
    t = _row_block_call(_tangent_body, x)
    p = _spmm_sc(t, srcp, dstp, valp)
    h1 = _row_block_call(_add_body, p[0], p[1])
    q = _spmm_sc(h1, srcp, dstp, valp)
    return _row_block_call(_final_body, h1, q[0], q[1])


# speedup vs baseline: 4.4843x; 4.4843x over previous
import functools

import jax
import jax.numpy as jnp
from jax import lax
from jax.experimental import pallas as pl
from jax.experimental.pallas import tpu as pltpu
from jax.experimental.pallas import tpu_sc as plsc

N = 10000
E = 320000
D = 128
EPS = 1e-7
MIN_NORM = 1e-15

NC = 2
NS = 16
K = 96
NCH = 108
EPT = NCH * K
EPAD = NC * NS * EPT
NSLOT = 3
NLSL = 6
RPT = 624
ZR = 16
NZC = RPT // ZR


def _spmm_body(mat_hbm, src_hbm, dst_hbm, val_hbm, out_hbm,
               acc, sidx, didx, vals, rows, zbuf,
               lsem0, lsem1, lsem2, lsem3, lsem4, lsem5,
               gsem0, gsem1, gsem2, ssem0, ssem1, ssem2, zsem):
    cid = lax.axis_index("c")
    sid = lax.axis_index("s")
    lsem = (lsem0, lsem1, lsem2, lsem3, lsem4, lsem5)
    gsem = (gsem0, gsem1, gsem2)
    ssem = (ssem0, ssem1, ssem2)

    tile = cid * NS + sid
    ebase = tile * EPT
    row0 = sid * RPT
    last = sid == NS - 1

    def issue_load(c, u):
        off = ebase + c * K
        pltpu.async_copy(src_hbm.at[pl.ds(off, K)], sidx.at[u], lsem[u])
        pltpu.async_copy(dst_hbm.at[pl.ds(off, K)], didx.at[u], lsem[u])
        pltpu.async_copy(val_hbm.at[pl.ds(off, K)], vals.at[u], lsem[u])

    def wait_load(c, u):
        off = ebase + c * K
        pltpu.make_async_copy(src_hbm.at[pl.ds(off, K)], sidx.at[u], lsem[u]).wait()
        pltpu.make_async_copy(dst_hbm.at[pl.ds(off, K)], didx.at[u], lsem[u]).wait()
        pltpu.make_async_copy(val_hbm.at[pl.ds(off, K)], vals.at[u], lsem[u]).wait()

    def issue_gather(u, s):
        pltpu.async_copy(mat_hbm.at[sidx.at[u]], rows.at[s], gsem[s])

    def wait_gather(u, s):
        pltpu.make_async_copy(mat_hbm.at[sidx.at[u]], rows.at[s], gsem[s]).wait()

    def issue_scatter(u, s):
        pltpu.async_copy(rows.at[s], acc.at[didx.at[u]], ssem[s], add=True)

    def wait_scatter(s):
        pltpu.make_async_copy(rows.at[s], acc.at[didx.at[0]], ssem[s]).wait()

    def scale(u, s):
        def group(g, carry):
            vv = vals[u, pl.ds(g * 16, 16)]
            for l in range(16):
                r = g * 16 + l
                sv = jnp.broadcast_to(vv[l], (16,))
                for q in range(D // 16):
                    sl = pl.ds(q * 16, 16)
                    rows[s, r, sl] = rows[s, r, sl] * sv
            return carry

        lax.fori_loop(0, K // 16, group, 0)

    z16 = jnp.zeros((16,), jnp.float32)
    for r in range(ZR):
        for q in range(D // 16):
            zbuf[r, pl.ds(q * 16, 16)] = z16
    zds = [pltpu.async_copy(zbuf, acc.at[pl.ds(row0 + k * ZR, ZR), :], zsem)
           for k in range(NZC)]

    for c in range(4):
        issue_load(c, c)
    for c in range(2):
        wait_load(c, c)
        issue_gather(c, c)

    for d in zds:
        d.wait()

    @pl.when(last)
    def _():
        pltpu.sync_copy(zbuf, acc.at[pl.ds(row0 + NZC * ZR, ZR), :])

    plsc.subcore_barrier()

    def round_(k, carry):
        t0 = k * NLSL
        for uu in range(NLSL):
            t = t0 + uu

            @pl.when(t + 4 < NCH)
            def _():
                issue_load(t + 4, (uu + 4) % NLSL)

            @pl.when(t + 2 < NCH)
            def _():
                if uu == 0:
                    @pl.when(k > 0)
                    def _():
                        wait_scatter((uu + 2) % NSLOT)
                else:
                    wait_scatter((uu + 2) % NSLOT)
                wait_load(t + 2, (uu + 2) % NLSL)
                issue_gather((uu + 2) % NLSL, (uu + 2) % NSLOT)

            wait_gather(uu, uu % NSLOT)
            scale(uu, uu % NSLOT)
            issue_scatter(uu, uu % NSLOT)
        return carry

    lax.fori_loop(0, NCH // NLSL, round_, 0)

    for s in range(NSLOT):
        wait_scatter(s)

    plsc.subcore_barrier()

    wds = [pltpu.async_copy(acc.at[pl.ds(row0 + k * ZR, ZR), :],
                            out_hbm.at[cid, pl.ds(row0 + k * ZR, ZR), :], zsem)
           for k in range(NZC)]

    @pl.when(last)
    def _():
        r0 = row0 + NZC * ZR
        pltpu.sync_copy(acc.at[pl.ds(r0, ZR), :], out_hbm.at[cid, pl.ds(r0, ZR), :])

    for d in wds:
        d.wait()


def _spmm_sc(mat, srcp, dstp, valp):
    mesh = plsc.VectorSubcoreMesh(
        core_axis_name="c", subcore_axis_name="s", num_cores=NC, num_subcores=NS)
    f = pl.kernel(
        _spmm_body,
        out_type=jax.ShapeDtypeStruct((NC, N, D), jnp.float32),
        mesh=mesh,
        scratch_types=[
            pltpu.VMEM_SHARED((N, D), jnp.float32),
            pltpu.VMEM((NLSL, K), jnp.int32),
            pltpu.VMEM((NLSL, K), jnp.int32),
            pltpu.VMEM((NLSL, K), jnp.float32),
            pltpu.VMEM((NSLOT, K, D), jnp.float32),
            pltpu.VMEM((ZR, D), jnp.float32),
        ] + [pltpu.SemaphoreType.DMA] * 13,
    )
    return f(mat, srcp, dstp, valp)


def _row_block_call(body, *args):
    rows = 1000
    grid = (N // rows,)
    spec = pl.BlockSpec((rows, D), lambda i: (i, 0))
    out_shape = jax.ShapeDtypeStruct((N, D), jnp.float32)
    return pl.pallas_call(
        body, out_shape=out_shape, grid=grid,
        in_specs=[spec] * len(args), out_specs=spec)(*args)


def _tangent_body(x_ref, o_ref):
    xb = x_ref[...]
    col = lax.broadcasted_iota(jnp.int32, xb.shape, 1)
    xm = jnp.where(col > 0, xb, 0.0)
    s = jnp.sum(xm * xm, axis=1, keepdims=True)
    t = jnp.sqrt(1.0 + s)
    theta = jnp.maximum(t, 1.0 + EPS)
    yn = jnp.maximum(jnp.sqrt(s), MIN_NORM)
    coef = jnp.log(theta + jnp.sqrt(theta * theta - 1.0)) / yn
    o_ref[...] = xm * coef


def _add_body(a_ref, b_ref, o_ref):
    o_ref[...] = a_ref[...] + b_ref[...]


def _final_body(h_ref, a_ref, b_ref, o_ref):
    u = h_ref[...] + a_ref[...] + b_ref[...]
    col = lax.broadcasted_iota(jnp.int32, u.shape, 1)
    um = jnp.where(col > 0, u, 0.0)
    s = jnp.sum(um * um, axis=1, keepdims=True)
    xn = jnp.maximum(jnp.sqrt(s), MIN_NORM)
    e = jnp.exp(xn)
    sinh = 0.5 * (e - 1.0 / e)
    sp = (sinh / xn) * um
    s2 = jnp.sum(sp * sp, axis=1, keepdims=True)
    t2 = jnp.sqrt(jnp.maximum(1.0 + s2, EPS))
    o_ref[...] = jnp.where(col > 0, sp, t2)


def kernel(x, edge_index, adj_values):
    pad = EPAD - E
    spread = jnp.arange(pad, dtype=jnp.int32) % N
    dstp = jnp.concatenate([edge_index[0], spread])
    srcp = jnp.concatenate([edge_index[1], spread])
    valp = jnp.pad(adj_values, (0, pad))
    t = _row_block_call(_tangent_body, x)
    p = _spmm_sc(t, srcp, dstp, valp)
    h1 = _row_block_call(_add_body, p[0], p[1])
    q = _spmm_sc(h1, srcp, dstp, valp)
    return _row_block_call(_final_body, h1, q[0], q[1])


# speedup vs baseline: 4.5404x; 1.0125x over previous
import functools

import jax
import jax.numpy as jnp
from jax import lax
from jax.experimental import pallas as pl
from jax.experimental.pallas import tpu as pltpu
from jax.experimental.pallas import tpu_sc as plsc

N = 10000
E = 320000
D = 128
EPS = 1e-7
MIN_NORM = 1e-15

NC = 2
NS = 16
K = 96
KH = K // 2
NCH = 108
EPT = NCH * K
EPAD = NC * NS * EPT
NSLOT = 3
NLSL = 6
RPT = 624
ZR = 16
NZC = RPT // ZR


def _spmm_body(mat_hbm, src_hbm, dst_hbm, val_hbm, out_hbm,
               acc, sidx, didx, vals, rows, zbuf,
               lsem0, lsem1, lsem2, lsem3, lsem4, lsem5,
               gsem0, gsem1, gsem2, ssem0, ssem1, ssem2, zsem):
    cid = lax.axis_index("c")
    sid = lax.axis_index("s")
    lsem = (lsem0, lsem1, lsem2, lsem3, lsem4, lsem5)
    gsem = (gsem0, gsem1, gsem2)
    ssem = (ssem0, ssem1, ssem2)

    tile = cid * NS + sid
    ebase = tile * EPT
    row0 = sid * RPT
    last = sid == NS - 1

    def issue_load(c, u):
        off = ebase + c * K
        pltpu.async_copy(src_hbm.at[pl.ds(off, K)], sidx.at[u], lsem[u])
        for h in range(2):
            pltpu.async_copy(dst_hbm.at[pl.ds(off + h * KH, KH)],
                             didx.at[u, h], lsem[u])
        pltpu.async_copy(val_hbm.at[pl.ds(off, K)], vals.at[u], lsem[u])

    def wait_load(c, u):
        off = ebase + c * K
        pltpu.make_async_copy(src_hbm.at[pl.ds(off, K)], sidx.at[u], lsem[u]).wait()
        for h in range(2):
            pltpu.make_async_copy(dst_hbm.at[pl.ds(off + h * KH, KH)],
                                  didx.at[u, h], lsem[u]).wait()
        pltpu.make_async_copy(val_hbm.at[pl.ds(off, K)], vals.at[u], lsem[u]).wait()

    def issue_gather(u, s):
        pltpu.async_copy(mat_hbm.at[sidx.at[u]], rows.at[s], gsem[s])

    def wait_gather(u, s):
        pltpu.make_async_copy(mat_hbm.at[sidx.at[u]], rows.at[s], gsem[s]).wait()

    def wait_scatter(s):
        for h in range(2):
            pltpu.make_async_copy(rows.at[s, pl.ds(0, KH)],
                                  acc.at[didx.at[0, 0]], ssem[s]).wait()

    def scale_and_scatter(u, s):
        for h in range(2):
            def group(g, carry):
                vv = vals[u, pl.ds(h * KH + g * 16, 16)]
                for l in range(16):
                    r = h * KH + g * 16 + l
                    sv = jnp.broadcast_to(vv[l], (16,))
                    for q in range(D // 16):
                        sl = pl.ds(q * 16, 16)
                        rows[s, r, sl] = rows[s, r, sl] * sv
                return carry

            lax.fori_loop(0, KH // 16, group, 0)
            pltpu.async_copy(rows.at[s, pl.ds(h * KH, KH)],
                             acc.at[didx.at[u, h]], ssem[s], add=True)

    z16 = jnp.zeros((16,), jnp.float32)
    for r in range(ZR):
        for q in range(D // 16):
            zbuf[r, pl.ds(q * 16, 16)] = z16
    zds = [pltpu.async_copy(zbuf, acc.at[pl.ds(row0 + k * ZR, ZR), :], zsem)
           for k in range(NZC)]

    for c in range(4):
        issue_load(c, c)
    for c in range(2):
        wait_load(c, c)
        issue_gather(c, c)

    for d in zds:
        d.wait()

    @pl.when(last)
    def _():
        pltpu.sync_copy(zbuf, acc.at[pl.ds(row0 + NZC * ZR, ZR), :])

    plsc.subcore_barrier()

    def round_(k, carry):
        t0 = k * NLSL
        for uu in range(NLSL):
            t = t0 + uu

            @pl.when(t + 4 < NCH)
            def _():
                issue_load(t + 4, (uu + 4) % NLSL)

            @pl.when(t + 2 < NCH)
            def _():
                if uu == 0:
                    @pl.when(k > 0)
                    def _():
                        wait_scatter((uu + 2) % NSLOT)
                else:
                    wait_scatter((uu + 2) % NSLOT)
                wait_load(t + 2, (uu + 2) % NLSL)
                issue_gather((uu + 2) % NLSL, (uu + 2) % NSLOT)

            wait_gather(uu, uu % NSLOT)
            scale_and_scatter(uu, uu % NSLOT)
        return carry

    lax.fori_loop(0, NCH // NLSL, round_, 0)

    for s in range(NSLOT):
        wait_scatter(s)

    plsc.subcore_barrier()

    wds = [pltpu.async_copy(acc.at[pl.ds(row0 + k * ZR, ZR), :],
                            out_hbm.at[cid, pl.ds(row0 + k * ZR, ZR), :], zsem)
           for k in range(NZC)]

    @pl.when(last)
    def _():
        r0 = row0 + NZC * ZR
        pltpu.sync_copy(acc.at[pl.ds(r0, ZR), :], out_hbm.at[cid, pl.ds(r0, ZR), :])

    for d in wds:
        d.wait()


def _spmm_sc(mat, srcp, dstp, valp):
    mesh = plsc.VectorSubcoreMesh(
        core_axis_name="c", subcore_axis_name="s", num_cores=NC, num_subcores=NS)
    f = pl.kernel(
        _spmm_body,
        out_type=jax.ShapeDtypeStruct((NC, N, D), jnp.float32),
        mesh=mesh,
        scratch_types=[
            pltpu.VMEM_SHARED((N, D), jnp.float32),
            pltpu.VMEM((NLSL, K), jnp.int32),
            pltpu.VMEM((NLSL, 2, KH), jnp.int32),
            pltpu.VMEM((NLSL, K), jnp.float32),
            pltpu.VMEM((NSLOT, K, D), jnp.float32),
            pltpu.VMEM((ZR, D), jnp.float32),
        ] + [pltpu.SemaphoreType.DMA] * 13,
    )
    return f(mat, srcp, dstp, valp)


def _row_block_call(body, *args):
    rows = 1000
    grid = (N // rows,)
    spec = pl.BlockSpec((rows, D), lambda i: (i, 0))
    out_shape = jax.ShapeDtypeStruct((N, D), jnp.float32)
    return pl.pallas_call(
        body, out_shape=out_shape, grid=grid,
        in_specs=[spec] * len(args), out_specs=spec)(*args)


def _tangent_body(x_ref, o_ref):
    xb = x_ref[...]
    col = lax.broadcasted_iota(jnp.int32, xb.shape, 1)
    xm = jnp.where(col > 0, xb, 0.0)
    s = jnp.sum(xm * xm, axis=1, keepdims=True)
    t = jnp.sqrt(1.0 + s)
    theta = jnp.maximum(t, 1.0 + EPS)
    yn = jnp.maximum(jnp.sqrt(s), MIN_NORM)
    coef = jnp.log(theta + jnp.sqrt(theta * theta - 1.0)) / yn
    o_ref[...] = xm * coef


def _add_body(a_ref, b_ref, o_ref):
    o_ref[...] = a_ref[...] + b_ref[...]


def _final_body(h_ref, a_ref, b_ref, o_ref):
    u = h_ref[...] + a_ref[...] + b_ref[...]
    col = lax.broadcasted_iota(jnp.int32, u.shape, 1)
    um = jnp.where(col > 0, u, 0.0)
    s = jnp.sum(um * um, axis=1, keepdims=True)
    xn = jnp.maximum(jnp.sqrt(s), MIN_NORM)
    e = jnp.exp(xn)
    sinh = 0.5 * (e - 1.0 / e)
    sp = (sinh / xn) * um
    s2 = jnp.sum(sp * sp, axis=1, keepdims=True)
    t2 = jnp.sqrt(jnp.maximum(1.0 + s2, EPS))
    o_ref[...] = jnp.where(col > 0, sp, t2)


def kernel(x, edge_index, adj_values):
    pad = EPAD - E
    spread = jnp.arange(pad, dtype=jnp.int32) % N
    dstp = jnp.concatenate([edge_index[0], spread])
    srcp = jnp.concatenate([edge_index[1], spread])
    valp = jnp.pad(adj_values, (0, pad))
    t = _row_block_call(_tangent_body, x)
    p = _spmm_sc(t, srcp, dstp, valp)
    h1 = _row_block_call(_add_body, p[0], p[1])
    q = _spmm_sc(h1, srcp, dstp, valp)
    return _row_block_call(_final_body, h1, q[0], q[1])


# speedup vs baseline: 4.6267x; 1.0190x over previous
import functools

import jax
import jax.numpy as jnp
from jax import lax
from jax.experimental import pallas as pl
from jax.experimental.pallas import tpu as pltpu
from jax.experimental.pallas import tpu_sc as plsc

N = 10000
E = 320000
D = 128
EPS = 1e-7
MIN_NORM = 1e-15

NC = 2
NS = 16
K = 96
KH = K // 2
NCH = 108
EPT = NCH * K
EPAD = NC * NS * EPT
NSLOT = 3
NLSL = 6
RPT = 624
ZR = 16
NZC = RPT // ZR


def _spmm_body(mat_hbm, src_hbm, dst_hbm, val_hbm, out_hbm,
               acc, sidx, didx, vals, rowsb, zbuf,
               lsem0, lsem1, lsem2, lsem3, lsem4, lsem5,
               gsem0, gsem1, gsem2, ssem0, ssem1, ssem2, zsem):
    cid = lax.axis_index("c")
    sid = lax.axis_index("s")
    lsem = (lsem0, lsem1, lsem2, lsem3, lsem4, lsem5)
    gsem = (gsem0, gsem1, gsem2)
    ssem = (ssem0, ssem1, ssem2)

    tile = cid * NS + sid
    ebase = tile * EPT
    row0 = sid * RPT
    last = sid == NS - 1

    def issue_load(c, u):
        off = ebase + c * K
        pltpu.async_copy(src_hbm.at[pl.ds(off, K)], sidx.at[u], lsem[u])
        for h in range(2):
            pltpu.async_copy(dst_hbm.at[pl.ds(off + h * KH, KH)],
                             didx.at[u, h], lsem[u])
        pltpu.async_copy(val_hbm.at[pl.ds(off, K)], vals.at[u], lsem[u])

    def wait_load(c, u):
        off = ebase + c * K
        pltpu.make_async_copy(src_hbm.at[pl.ds(off, K)], sidx.at[u], lsem[u]).wait()
        for h in range(2):
            pltpu.make_async_copy(dst_hbm.at[pl.ds(off + h * KH, KH)],
                                  didx.at[u, h], lsem[u]).wait()
        pltpu.make_async_copy(val_hbm.at[pl.ds(off, K)], vals.at[u], lsem[u]).wait()

    def issue_gather(u, s):
        pltpu.async_copy(mat_hbm.at[sidx.at[u]], rowsb.at[s], gsem[s])

    def wait_gather(u, s):
        pltpu.make_async_copy(mat_hbm.at[sidx.at[u]], rowsb.at[s], gsem[s]).wait()

    def wait_scatter(s):
        for h in range(2):
            pltpu.make_async_copy(rowsb.at[s, pl.ds(0, KH)],
                                  acc.at[didx.at[0, 0]], ssem[s]).wait()

    def scale_and_scatter(u, s):
        for h in range(2):
            def group(g, carry):
                vv = vals[u, pl.ds(h * KH + g * 16, 16)]
                for l in range(16):
                    r = h * KH + g * 16 + l
                    sv = jnp.broadcast_to(vv[l], (16,))
                    for q in range(D // 16):
                        sl = pl.ds(q * 16, 16)
                        rowsb[s, r, sl] = rowsb[s, r, sl] * sv
                return carry

            lax.fori_loop(0, KH // 16, group, 0)
            pltpu.async_copy(rowsb.at[s, pl.ds(h * KH, KH)],
                             acc.at[didx.at[u, h]], ssem[s], add=True)

    z16 = jnp.zeros((16,), jnp.float32)
    for r in range(ZR):
        for q in range(D // 16):
            zbuf[r, pl.ds(q * 16, 16)] = z16
    zds = [pltpu.async_copy(zbuf, acc.at[pl.ds(row0 + k * ZR, ZR), :], zsem)
           for k in range(NZC)]

    for c in range(4):
        issue_load(c, c)
    for c in range(2):
        wait_load(c, c)
        issue_gather(c, c)

    for d in zds:
        d.wait()

    @pl.when(last)
    def _():
        pltpu.sync_copy(zbuf, acc.at[pl.ds(row0 + NZC * ZR, ZR), :])

    plsc.subcore_barrier()

    def round_(k, carry):
        t0 = k * NLSL
        for uu in range(NLSL):
            t = t0 + uu

            @pl.when(t + 4 < NCH)
            def _():
                issue_load(t + 4, (uu + 4) % NLSL)

            @pl.when(t + 2 < NCH)
            def _():
                if uu == 0:
                    @pl.when(k > 0)
                    def _():
                        wait_scatter((uu + 2) % NSLOT)
                else:
                    wait_scatter((uu + 2) % NSLOT)
                wait_load(t + 2, (uu + 2) % NLSL)
                issue_gather((uu + 2) % NLSL, (uu + 2) % NSLOT)

            wait_gather(uu, uu % NSLOT)
            scale_and_scatter(uu, uu % NSLOT)
        return carry

    lax.fori_loop(0, NCH // NLSL, round_, 0)

    for s in range(NSLOT):
        wait_scatter(s)

    plsc.subcore_barrier()

    wds = [pltpu.async_copy(acc.at[pl.ds(row0 + k * ZR, ZR), :],
                            out_hbm.at[cid, pl.ds(row0 + k * ZR, ZR), :], zsem)
           for k in range(NZC)]

    @pl.when(last)
    def _():
        r0 = row0 + NZC * ZR
        pltpu.sync_copy(acc.at[pl.ds(r0, ZR), :], out_hbm.at[cid, pl.ds(r0, ZR), :])

    for d in wds:
        d.wait()


def _spmm_sc(mat, srcp, dstp, valp):
    mesh = plsc.VectorSubcoreMesh(
        core_axis_name="c", subcore_axis_name="s", num_cores=NC, num_subcores=NS)
    f = pl.kernel(
        _spmm_body,
        out_type=jax.ShapeDtypeStruct((NC, N, D), jnp.float32),
        mesh=mesh,
        scratch_types=[
            pltpu.VMEM_SHARED((N, D), jnp.float32),
            pltpu.VMEM((NLSL, K), jnp.int32),
            pltpu.VMEM((NLSL, 2, KH), jnp.int32),
            pltpu.VMEM((NLSL, K), jnp.float32),
            pltpu.VMEM((NSLOT, K, D), jnp.float32),
            pltpu.VMEM((ZR, D), jnp.float32),
        ] + [pltpu.SemaphoreType.DMA] * 13,
    )
    return f(mat, srcp, dstp, valp)


def _row_block_call(body, *args):
    rows = 2000
    grid = (N // rows,)
    spec = pl.BlockSpec((rows, D), lambda i: (i, 0))
    out_shape = jax.ShapeDtypeStruct((N, D), jnp.float32)
    return pl.pallas_call(
        body, out_shape=out_shape, grid=grid,
        in_specs=[spec] * len(args), out_specs=spec)(*args)


def _tangent_body(x_ref, o_ref):
    xb = x_ref[...]
    col = lax.broadcasted_iota(jnp.int32, xb.shape, 1)
    xm = jnp.where(col > 0, xb, 0.0)
    s = jnp.sum(xm * xm, axis=1, keepdims=True)
    t = jnp.sqrt(1.0 + s)
    theta = jnp.maximum(t, 1.0 + EPS)
    yn = jnp.maximum(jnp.sqrt(s), MIN_NORM)
    coef = jnp.log(theta + jnp.sqrt(theta * theta - 1.0)) / yn
    o_ref[...] = xm * coef


def _add_body(a_ref, b_ref, o_ref):
    o_ref[...] = a_ref[...] + b_ref[...]


def _final_body(h_ref, a_ref, b_ref, o_ref):
    u = h_ref[...] + a_ref[...] + b_ref[...]
    col = lax.broadcasted_iota(jnp.int32, u.shape, 1)
    um = jnp.where(col > 0, u, 0.0)
    s = jnp.sum(um * um, axis=1, keepdims=True)
    xn = jnp.maximum(jnp.sqrt(s), MIN_NORM)
    e = jnp.exp(xn)
    sinh = 0.5 * (e - 1.0 / e)
    sp = (sinh / xn) * um
    s2 = jnp.sum(sp * sp, axis=1, keepdims=True)
    t2 = jnp.sqrt(jnp.maximum(1.0 + s2, EPS))
    o_ref[...] = jnp.where(col > 0, sp, t2)


def kernel(x, edge_index, adj_values):
    pad = EPAD - E
    spread = jnp.arange(pad, dtype=jnp.int32) % N
    dstp = jnp.concatenate([edge_index[0], spread])
    srcp = jnp.concatenate([edge_index[1], spread])
    valp = jnp.pad(adj_values, (0, pad))
    t = _row_block_call(_tangent_body, x)
    p = _spmm_sc(t, srcp, dstp, valp)
    h1 = _row_block_call(_add_body, p[0], p[1])
    q = _spmm_sc(h1, srcp, dstp, valp)
    return _row_block_call(_final_body, h1, q[0], q[1])


# speedup vs baseline: 5.1275x; 1.1083x over previous
import functools

import jax
import jax.numpy as jnp
from jax import lax
from jax.experimental import pallas as pl
from jax.experimental.pallas import tpu as pltpu
from jax.experimental.pallas import tpu_sc as plsc

N = 10000
E = 320000
D = 128
EPS = 1e-7
MIN_NORM = 1e-15

NC = 2
NS = 16
EPT = E // (NC * NS)
K = 96
KH = K // 2
NCH = EPT // K
NCHR = (NCH // 6) * 6
TAIL = EPT - NCH * K
NSLOT = 3
NLSL = 6
RPT = 624
ZR = 16
NZC = RPT // ZR


def _spmm_body(mat_hbm, ef_hbm, val_hbm, out_hbm,
               acc, sidx, didx, vals, rowsb, zbuf, tidx, tval,
               lsem0, lsem1, lsem2, lsem3, lsem4, lsem5,
               gsem0, gsem1, gsem2, ssem0, ssem1, ssem2, zsem, tsem):
    cid = lax.axis_index("c")
    sid = lax.axis_index("s")
    lsem = (lsem0, lsem1, lsem2, lsem3, lsem4, lsem5)
    gsem = (gsem0, gsem1, gsem2)
    ssem = (ssem0, ssem1, ssem2)

    tile = cid * NS + sid
    ebase = tile * EPT
    row0 = sid * RPT
    last = sid == NS - 1

    def issue_load(c, u):
        off = ebase + c * K
        pltpu.async_copy(ef_hbm.at[pl.ds(E + off, K)], sidx.at[u], lsem[u])
        for h in range(2):
            pltpu.async_copy(ef_hbm.at[pl.ds(off + h * KH, KH)],
                             didx.at[u, h], lsem[u])
        pltpu.async_copy(val_hbm.at[pl.ds(off, K)], vals.at[u], lsem[u])

    def wait_load(c, u):
        off = ebase + c * K
        pltpu.make_async_copy(ef_hbm.at[pl.ds(E + off, K)], sidx.at[u], lsem[u]).wait()
        for h in range(2):
            pltpu.make_async_copy(ef_hbm.at[pl.ds(off + h * KH, KH)],
                                  didx.at[u, h], lsem[u]).wait()
        pltpu.make_async_copy(val_hbm.at[pl.ds(off, K)], vals.at[u], lsem[u]).wait()

    def issue_gather(u, s):
        pltpu.async_copy(mat_hbm.at[sidx.at[u]], rowsb.at[s], gsem[s])

    def wait_gather(u, s):
        pltpu.make_async_copy(mat_hbm.at[sidx.at[u]], rowsb.at[s], gsem[s]).wait()

    def wait_scatter(s):
        for h in range(2):
            pltpu.make_async_copy(rowsb.at[s, pl.ds(0, KH)],
                                  acc.at[didx.at[0, 0]], ssem[s]).wait()

    def scale_and_scatter(u, s):
        for h in range(2):
            def group(g, carry):
                vv = vals[u, pl.ds(h * KH + g * 16, 16)]
                for l in range(16):
                    r = h * KH + g * 16 + l
                    sv = jnp.broadcast_to(vv[l], (16,))
                    for q in range(D // 16):
                        sl = pl.ds(q * 16, 16)
                        rowsb[s, r, sl] = rowsb[s, r, sl] * sv
                return carry

            lax.fori_loop(0, KH // 16, group, 0)
            pltpu.async_copy(rowsb.at[s, pl.ds(h * KH, KH)],
                             acc.at[didx.at[u, h]], ssem[s], add=True)

    z16 = jnp.zeros((16,), jnp.float32)
    for r in range(ZR):
        for q in range(D // 16):
            zbuf[r, pl.ds(q * 16, 16)] = z16
    zds = [pltpu.async_copy(zbuf, acc.at[pl.ds(row0 + k * ZR, ZR), :], zsem)
           for k in range(NZC)]

    toff = ebase + NCH * K
    pltpu.async_copy(ef_hbm.at[pl.ds(E + toff, TAIL)], tidx.at[0], tsem)
    pltpu.async_copy(ef_hbm.at[pl.ds(toff, TAIL)], tidx.at[1], tsem)
    pltpu.async_copy(val_hbm.at[pl.ds(toff, TAIL)], tval, tsem)
    for c in range(4):
        issue_load(c, c)
    pltpu.make_async_copy(ef_hbm.at[pl.ds(E + toff, TAIL)], tidx.at[0], tsem).wait()
    pltpu.make_async_copy(ef_hbm.at[pl.ds(toff, TAIL)], tidx.at[1], tsem).wait()
    pltpu.make_async_copy(val_hbm.at[pl.ds(toff, TAIL)], tval, tsem).wait()
    for c in range(2):
        wait_load(c, c)
        issue_gather(c, c)

    for d in zds:
        d.wait()

    @pl.when(last)
    def _():
        pltpu.sync_copy(zbuf, acc.at[pl.ds(row0 + NZC * ZR, ZR), :])

    plsc.subcore_barrier()

    def round_(k, carry):
        t0 = k * NLSL
        for uu in range(NLSL):
            t = t0 + uu

            @pl.when(t + 4 < NCH)
            def _():
                issue_load(t + 4, (uu + 4) % NLSL)

            @pl.when(t + 2 < NCH)
            def _():
                if uu == 0:
                    @pl.when(k > 0)
                    def _():
                        wait_scatter((uu + 2) % NSLOT)
                else:
                    wait_scatter((uu + 2) % NSLOT)
                wait_load(t + 2, (uu + 2) % NLSL)
                issue_gather((uu + 2) % NLSL, (uu + 2) % NSLOT)

            wait_gather(uu, uu % NSLOT)
            scale_and_scatter(uu, uu % NSLOT)
        return carry

    lax.fori_loop(0, NCHR // NLSL, round_, 0)

    for t in range(NCHR, NCH):
        wait_gather(t % NLSL, t % NSLOT)
        scale_and_scatter(t % NLSL, t % NSLOT)

    for s in range(NSLOT):
        wait_scatter(s)

    pltpu.async_copy(mat_hbm.at[tidx.at[0]], rowsb.at[0, pl.ds(0, TAIL)],
                     gsem0).wait()
    tvv = tval[...]
    for l in range(TAIL):
        sv = jnp.broadcast_to(tvv[l], (16,))
        for q in range(D // 16):
            sl = pl.ds(q * 16, 16)
            rowsb[0, l, sl] = rowsb[0, l, sl] * sv
    pltpu.sync_copy(rowsb.at[0, pl.ds(0, TAIL)], acc.at[tidx.at[1]], add=True)

    plsc.subcore_barrier()

    wds = [pltpu.async_copy(acc.at[pl.ds(row0 + k * ZR, ZR), :],
                            out_hbm.at[cid, pl.ds(row0 + k * ZR, ZR), :], zsem)
           for k in range(NZC)]

    @pl.when(last)
    def _():
        r0 = row0 + NZC * ZR
        pltpu.sync_copy(acc.at[pl.ds(r0, ZR), :], out_hbm.at[cid, pl.ds(r0, ZR), :])

    for d in wds:
        d.wait()


def _spmm_sc(mat, ef, val):
    mesh = plsc.VectorSubcoreMesh(
        core_axis_name="c", subcore_axis_name="s", num_cores=NC, num_subcores=NS)
    f = pl.kernel(
        _spmm_body,
        out_type=jax.ShapeDtypeStruct((NC, N, D), jnp.float32),
        mesh=mesh,
        scratch_types=[
            pltpu.VMEM_SHARED((N, D), jnp.float32),
            pltpu.VMEM((NLSL, K), jnp.int32),
            pltpu.VMEM((NLSL, 2, KH), jnp.int32),
            pltpu.VMEM((NLSL, K), jnp.float32),
            pltpu.VMEM((NSLOT, K, D), jnp.float32),
            pltpu.VMEM((ZR, D), jnp.float32),
            pltpu.VMEM((2, TAIL), jnp.int32),
            pltpu.VMEM((TAIL,), jnp.float32),
        ] + [pltpu.SemaphoreType.DMA] * 14,
    )
    return f(mat, ef, val)


_ROWS = 2000


def _tangent_body(x_ref, o_ref):
    xb = x_ref[...]
    col = lax.broadcasted_iota(jnp.int32, xb.shape, 1)
    xm = jnp.where(col > 0, xb, 0.0)
    s = jnp.sum(xm * xm, axis=1, keepdims=True)
    t = jnp.sqrt(1.0 + s)
    theta = jnp.maximum(t, 1.0 + EPS)
    yn = jnp.maximum(jnp.sqrt(s), MIN_NORM)
    coef = jnp.log(theta + jnp.sqrt(theta * theta - 1.0)) / yn
    o_ref[...] = xm * coef


def _add_body(p_ref, o_ref):
    o_ref[...] = p_ref[0] + p_ref[1]


def _final_body(h_ref, q_ref, o_ref):
    u = h_ref[...] + (q_ref[0] + q_ref[1])
    col = lax.broadcasted_iota(jnp.int32, u.shape, 1)
    um = jnp.where(col > 0, u, 0.0)
    s = jnp.sum(um * um, axis=1, keepdims=True)
    xn = jnp.maximum(jnp.sqrt(s), MIN_NORM)
    e = jnp.exp(xn)
    sinh = 0.5 * (e - 1.0 / e)
    sp = (sinh / xn) * um
    s2 = jnp.sum(sp * sp, axis=1, keepdims=True)
    t2 = jnp.sqrt(jnp.maximum(1.0 + s2, EPS))
    o_ref[...] = jnp.where(col > 0, sp, t2)


_SPEC2 = pl.BlockSpec((_ROWS, D), lambda i: (i, 0))
_SPEC3 = pl.BlockSpec((NC, _ROWS, D), lambda i: (0, i, 0))
_OUT = jax.ShapeDtypeStruct((N, D), jnp.float32)
_GRID = (N // _ROWS,)


def _tangent_tc(x):
    return pl.pallas_call(_tangent_body, out_shape=_OUT, grid=_GRID,
                          in_specs=[_SPEC2], out_specs=_SPEC2)(x)


def _add_tc(p):
    return pl.pallas_call(_add_body, out_shape=_OUT, grid=_GRID,
                          in_specs=[_SPEC3], out_specs=_SPEC2)(p)


def _final_tc(h1, q):
    return pl.pallas_call(_final_body, out_shape=_OUT, grid=_GRID,
                          in_specs=[_SPEC2, _SPEC3], out_specs=_SPEC2)(h1, q)


def kernel(x, edge_index, adj_values):
    ef = edge_index.reshape(2 * E)
    t = _tangent_tc(x)
    p = _spmm_sc(t, ef, adj_values)
    h1 = _add_tc(p)
    q = _spmm_sc(h1, ef, adj_values)
    return _final_tc(h1, q)


# speedup vs baseline: 5.1698x; 1.0082x over previous
import functools

import jax
import jax.numpy as jnp
from jax import lax
from jax.experimental import pallas as pl
from jax.experimental.pallas import tpu as pltpu
from jax.experimental.pallas import tpu_sc as plsc

N = 10000
E = 320000
D = 128
EPS = 1e-7
MIN_NORM = 1e-15

NC = 2
NS = 16
EPT = E // (NC * NS)
K = 96
KH = K // 2
NCH = EPT // K
NCHR = (NCH // 6) * 6
TAIL = EPT - NCH * K
NSLOT = 3
NLSL = 6
RPT = 624
ZR = 16
NZC = RPT // ZR


def _spmm_body(mat_hbm, ef_hbm, val_hbm, out_hbm,
               acc, sidx, didx, vals, rowsb, zbuf, tidx, tval,
               lsem0, lsem1, lsem2, lsem3, lsem4, lsem5,
               gsem0, gsem1, gsem2, ssem0, ssem1, ssem2, zsem, tsem):
    cid = lax.axis_index("c")
    sid = lax.axis_index("s")
    lsem = (lsem0, lsem1, lsem2, lsem3, lsem4, lsem5)
    gsem = (gsem0, gsem1, gsem2)
    ssem = (ssem0, ssem1, ssem2)

    tile = cid * NS + sid
    ebase = tile * EPT
    row0 = sid * RPT
    last = sid == NS - 1

    def issue_load(c, u):
        off = ebase + c * K
        pltpu.async_copy(ef_hbm.at[1, pl.ds(off, K)], sidx.at[u], lsem[u])
        for h in range(2):
            pltpu.async_copy(ef_hbm.at[0, pl.ds(off + h * KH, KH)],
                             didx.at[u, h], lsem[u])
        pltpu.async_copy(val_hbm.at[pl.ds(off, K)], vals.at[u], lsem[u])

    def wait_load(c, u):
        off = ebase + c * K
        pltpu.make_async_copy(ef_hbm.at[1, pl.ds(off, K)], sidx.at[u], lsem[u]).wait()
        for h in range(2):
            pltpu.make_async_copy(ef_hbm.at[0, pl.ds(off + h * KH, KH)],
                                  didx.at[u, h], lsem[u]).wait()
        pltpu.make_async_copy(val_hbm.at[pl.ds(off, K)], vals.at[u], lsem[u]).wait()

    def issue_gather(u, s):
        pltpu.async_copy(mat_hbm.at[sidx.at[u]], rowsb.at[s], gsem[s])

    def wait_gather(u, s):
        pltpu.make_async_copy(mat_hbm.at[sidx.at[u]], rowsb.at[s], gsem[s]).wait()

    def wait_scatter(s):
        for h in range(2):
            pltpu.make_async_copy(rowsb.at[s, pl.ds(0, KH)],
                                  acc.at[didx.at[0, 0]], ssem[s]).wait()

    def scale_and_scatter(u, s):
        for h in range(2):
            def group(g, carry):
                vv = vals[u, pl.ds(h * KH + g * 16, 16)]
                for l in range(16):
                    r = h * KH + g * 16 + l
                    sv = jnp.broadcast_to(vv[l], (16,))
                    for q in range(D // 16):
                        sl = pl.ds(q * 16, 16)
                        rowsb[s, r, sl] = rowsb[s, r, sl] * sv
                return carry

            lax.fori_loop(0, KH // 16, group, 0)
            pltpu.async_copy(rowsb.at[s, pl.ds(h * KH, KH)],
                             acc.at[didx.at[u, h]], ssem[s], add=True)

    z16 = jnp.zeros((16,), jnp.float32)
    for r in range(ZR):
        for q in range(D // 16):
            zbuf[r, pl.ds(q * 16, 16)] = z16
    zds = [pltpu.async_copy(zbuf, acc.at[pl.ds(row0 + k * ZR, ZR), :], zsem)
           for k in range(NZC)]

    toff = ebase + NCH * K
    pltpu.async_copy(ef_hbm.at[1, pl.ds(toff, TAIL)], tidx.at[0], tsem)
    pltpu.async_copy(ef_hbm.at[0, pl.ds(toff, TAIL)], tidx.at[1], tsem)
    pltpu.async_copy(val_hbm.at[pl.ds(toff, TAIL)], tval, tsem)
    for c in range(4):
        issue_load(c, c)
    pltpu.make_async_copy(ef_hbm.at[1, pl.ds(toff, TAIL)], tidx.at[0], tsem).wait()
    pltpu.make_async_copy(ef_hbm.at[0, pl.ds(toff, TAIL)], tidx.at[1], tsem).wait()
    pltpu.make_async_copy(val_hbm.at[pl.ds(toff, TAIL)], tval, tsem).wait()
    for c in range(2):
        wait_load(c, c)
        issue_gather(c, c)

    for d in zds:
        d.wait()

    @pl.when(last)
    def _():
        pltpu.sync_copy(zbuf, acc.at[pl.ds(row0 + NZC * ZR, ZR), :])

    plsc.subcore_barrier()

    def round_(k, carry):
        t0 = k * NLSL
        for uu in range(NLSL):
            t = t0 + uu

            @pl.when(t + 4 < NCH)
            def _():
                issue_load(t + 4, (uu + 4) % NLSL)

            @pl.when(t + 2 < NCH)
            def _():
                if uu == 0:
                    @pl.when(k > 0)
                    def _():
                        wait_scatter((uu + 2) % NSLOT)
                else:
                    wait_scatter((uu + 2) % NSLOT)
                wait_load(t + 2, (uu + 2) % NLSL)
                issue_gather((uu + 2) % NLSL, (uu + 2) % NSLOT)

            wait_gather(uu, uu % NSLOT)
            scale_and_scatter(uu, uu % NSLOT)
        return carry

    lax.fori_loop(0, NCHR // NLSL, round_, 0)

    for t in range(NCHR, NCH):
        wait_gather(t % NLSL, t % NSLOT)
        scale_and_scatter(t % NLSL, t % NSLOT)

    for s in range(NSLOT):
        wait_scatter(s)

    pltpu.async_copy(mat_hbm.at[tidx.at[0]], rowsb.at[0, pl.ds(0, TAIL)],
                     gsem0).wait()
    tvv = tval[...]
    for l in range(TAIL):
        sv = jnp.broadcast_to(tvv[l], (16,))
        for q in range(D // 16):
            sl = pl.ds(q * 16, 16)
            rowsb[0, l, sl] = rowsb[0, l, sl] * sv
    pltpu.sync_copy(rowsb.at[0, pl.ds(0, TAIL)], acc.at[tidx.at[1]], add=True)

    plsc.subcore_barrier()

    wds = [pltpu.async_copy(acc.at[pl.ds(row0 + k * ZR, ZR), :],
                            out_hbm.at[cid, pl.ds(row0 + k * ZR, ZR), :], zsem)
           for k in range(NZC)]

    @pl.when(last)
    def _():
        r0 = row0 + NZC * ZR
        pltpu.sync_copy(acc.at[pl.ds(r0, ZR), :], out_hbm.at[cid, pl.ds(r0, ZR), :])

    for d in wds:
        d.wait()


def _spmm_sc(mat, ef, val):
    mesh = plsc.VectorSubcoreMesh(
        core_axis_name="c", subcore_axis_name="s", num_cores=NC, num_subcores=NS)
    f = pl.kernel(
        _spmm_body,
        out_type=jax.ShapeDtypeStruct((NC, N, D), jnp.float32),
        mesh=mesh,
        compiler_params=pltpu.CompilerParams(use_tc_tiling_on_sc=False),
        scratch_types=[
            pltpu.VMEM_SHARED((N, D), jnp.float32),
            pltpu.VMEM((NLSL, K), jnp.int32),
            pltpu.VMEM((NLSL, 2, KH), jnp.int32),
            pltpu.VMEM((NLSL, K), jnp.float32),
            pltpu.VMEM((NSLOT, K, D), jnp.float32),
            pltpu.VMEM((ZR, D), jnp.float32),
            pltpu.VMEM((2, TAIL), jnp.int32),
            pltpu.VMEM((TAIL,), jnp.float32),
        ] + [pltpu.SemaphoreType.DMA] * 14,
    )
    return f(mat, ef, val)


_ROWS = 2000


def _tangent_body(x_ref, o_ref):
    xb = x_ref[...]
    col = lax.broadcasted_iota(jnp.int32, xb.shape, 1)
    xm = jnp.where(col > 0, xb, 0.0)
    s = jnp.sum(xm * xm, axis=1, keepdims=True)
    t = jnp.sqrt(1.0 + s)
    theta = jnp.maximum(t, 1.0 + EPS)
    yn = jnp.maximum(jnp.sqrt(s), MIN_NORM)
    coef = jnp.log(theta + yn) / yn
    o_ref[...] = xm * coef


def _add_body(p_ref, o_ref):
    o_ref[...] = p_ref[0] + p_ref[1]


def _final_body(h_ref, q_ref, o_ref):
    u = h_ref[...] + (q_ref[0] + q_ref[1])
    col = lax.broadcasted_iota(jnp.int32, u.shape, 1)
    um = jnp.where(col > 0, u, 0.0)
    s = jnp.sum(um * um, axis=1, keepdims=True)
    xn = jnp.maximum(jnp.sqrt(s), MIN_NORM)
    e = jnp.exp(xn)
    sinh = 0.5 * (e - 1.0 / e)
    sp = (sinh / xn) * um
    s2 = jnp.sum(sp * sp, axis=1, keepdims=True)
    t2 = jnp.sqrt(jnp.maximum(1.0 + s2, EPS))
    o_ref[...] = jnp.where(col > 0, sp, t2)


_SPEC2 = pl.BlockSpec((_ROWS, D), lambda i: (i, 0))
_SPEC3 = pl.BlockSpec((NC, _ROWS, D), lambda i: (0, i, 0))
_OUT = jax.ShapeDtypeStruct((N, D), jnp.float32)
_GRID = (N // _ROWS,)


def _tangent_tc(x):
    return pl.pallas_call(_tangent_body, out_shape=_OUT, grid=_GRID,
                          in_specs=[_SPEC2], out_specs=_SPEC2)(x)


def _add_tc(p):
    return pl.pallas_call(_add_body, out_shape=_OUT, grid=_GRID,
                          in_specs=[_SPEC3], out_specs=_SPEC2)(p)


def _final_tc(h1, q):
    return pl.pallas_call(_final_body, out_shape=_OUT, grid=_GRID,
                          in_specs=[_SPEC2, _SPEC3], out_specs=_SPEC2)(h1, q)


def kernel(x, edge_index, adj_values):
    t = _tangent_tc(x)
    p = _spmm_sc(t, edge_index, adj_values)
    h1 = _add_tc(p)
    q = _spmm_sc(h1, edge_index, adj_values)
    return _final_tc(h1, q)
